# initial kernel scaffold (unmeasured)
import jax
import jax.numpy as jnp
from jax import lax
from jax.experimental import pallas as pl
from jax.experimental.pallas import tpu as pltpu

N_DEV = 16


def kernel(x, w_mat, scale_x, scale_w):
    m, _ = x.shape
    _, n = w_mat.shape
    ch = m // N_DEV

    def body(x_ref, w_ref, sx_ref, sw_ref, out_ref, comm_ref,
             rs_send, rs_recv, ag_send, ag_recv):
        my = lax.axis_index("i")
        left = (my - 1 + N_DEV) % N_DEV
        right = (my + 1) % N_DEV

        barrier = pltpu.get_barrier_semaphore()
        pl.semaphore_signal(barrier, inc=1, device_id=(left,),
                            device_id_type=pl.DeviceIdType.MESH)
        pl.semaphore_wait(barrier, 1)

        out_ref[...] = jnp.dot(x_ref[...], w_ref[...],
                               preferred_element_type=jnp.float32)

        for s in range(N_DEV - 1):
            send_c = (my - s + N_DEV) % N_DEV
            rdma = pltpu.make_async_remote_copy(
                src_ref=out_ref.at[pl.ds(send_c * ch, ch), :],
                dst_ref=comm_ref.at[s],
                send_sem=rs_send.at[s],
                recv_sem=rs_recv.at[s],
                device_id=(right,),
                device_id_type=pl.DeviceIdType.MESH,
            )
            rdma.start()
            rdma.wait()
            recv_c = (my - s - 1 + N_DEV) % N_DEV
            cur = pl.load(out_ref, (pl.ds(recv_c * ch, ch), slice(None)))
            pl.store(out_ref, (pl.ds(recv_c * ch, ch), slice(None)),
                     cur + comm_ref[s])

        own = (my + 1) % N_DEV
        scale = sx_ref[0, 0] * sw_ref[0, 0]
        red = pl.load(out_ref, (pl.ds(own * ch, ch), slice(None)))
        pl.store(out_ref, (pl.ds(own * ch, ch), slice(None)),
                 jnp.maximum(red * scale, 0.0))

        for s in range(N_DEV - 1):
            send_c = (my + 1 - s + N_DEV) % N_DEV
            rdma = pltpu.make_async_remote_copy(
                src_ref=out_ref.at[pl.ds(send_c * ch, ch), :],
                dst_ref=out_ref.at[pl.ds(send_c * ch, ch), :],
                send_sem=ag_send.at[s],
                recv_sem=ag_recv.at[s],
                device_id=(right,),
                device_id_type=pl.DeviceIdType.MESH,
            )
            rdma.start()
            rdma.wait()

    return pl.pallas_call(
        body,
        out_shape=jax.ShapeDtypeStruct((m, n), jnp.float32),
        in_specs=[
            pl.BlockSpec(memory_space=pltpu.VMEM),
            pl.BlockSpec(memory_space=pltpu.VMEM),
            pl.BlockSpec(memory_space=pltpu.SMEM),
            pl.BlockSpec(memory_space=pltpu.SMEM),
        ],
        out_specs=pl.BlockSpec(memory_space=pltpu.VMEM),
        scratch_shapes=[
            pltpu.VMEM((N_DEV - 1, ch, n), jnp.float32),
            pltpu.SemaphoreType.DMA((N_DEV - 1,)),
            pltpu.SemaphoreType.DMA((N_DEV - 1,)),
            pltpu.SemaphoreType.DMA((N_DEV - 1,)),
            pltpu.SemaphoreType.DMA((N_DEV - 1,)),
        ],
        compiler_params=pltpu.CompilerParams(collective_id=0),
    )(x, w_mat, scale_x.reshape(1, 1), scale_w.reshape(1, 1))


# baseline (device time: 778201 ns/iter reference)
import jax
import jax.numpy as jnp
from jax import lax
from jax.experimental import pallas as pl
from jax.experimental.pallas import tpu as pltpu

N_DEV = 16


def kernel(x, w_mat, scale_x, scale_w):
    m, _ = x.shape
    _, n = w_mat.shape
    ch = m // N_DEV

    NSLOTS = 4

    def body(x_ref, w_ref, sx_ref, sw_ref, out_ref, comm_ref,
             rs_send, rs_recv, ag_send, ag_recv, credit_sem):
        my = lax.axis_index("i")
        left = (my - 1 + N_DEV) % N_DEV
        right = (my + 1) % N_DEV

        barrier = pltpu.get_barrier_semaphore()
        pl.semaphore_signal(barrier, inc=1, device_id=(left,),
                            device_id_type=pl.DeviceIdType.MESH)
        pl.semaphore_wait(barrier, 1)

        out_ref[...] = jnp.dot(x_ref[...].astype(jnp.bfloat16),
                               w_ref[...].astype(jnp.bfloat16),
                               preferred_element_type=jnp.float32)

        for s in range(N_DEV - 1):
            slot = s % NSLOTS
            if s >= NSLOTS:
                pl.semaphore_wait(credit_sem, 1)
            send_c = (my - s + N_DEV) % N_DEV
            rdma = pltpu.make_async_remote_copy(
                src_ref=out_ref.at[pl.ds(send_c * ch, ch), :],
                dst_ref=comm_ref.at[slot],
                send_sem=rs_send.at[s],
                recv_sem=rs_recv.at[s],
                device_id=(right,),
                device_id_type=pl.DeviceIdType.MESH,
            )
            rdma.start()
            rdma.wait()
            recv_c = (my - s - 1 + N_DEV) % N_DEV
            out_ref[pl.ds(recv_c * ch, ch), :] = (
                out_ref[pl.ds(recv_c * ch, ch), :] + comm_ref[slot]
            )
            if s < (N_DEV - 1) - NSLOTS:
                pl.semaphore_signal(credit_sem, inc=1, device_id=(left,),
                                    device_id_type=pl.DeviceIdType.MESH)

        own = (my + 1) % N_DEV
        scale = sx_ref[0, 0] * sw_ref[0, 0]
        red = out_ref[pl.ds(own * ch, ch), :]
        out_ref[pl.ds(own * ch, ch), :] = jnp.maximum(red * scale, 0.0)

        for s in range(N_DEV - 1):
            send_c = (my + 1 - s + N_DEV) % N_DEV
            rdma = pltpu.make_async_remote_copy(
                src_ref=out_ref.at[pl.ds(send_c * ch, ch), :],
                dst_ref=out_ref.at[pl.ds(send_c * ch, ch), :],
                send_sem=ag_send.at[s],
                recv_sem=ag_recv.at[s],
                device_id=(right,),
                device_id_type=pl.DeviceIdType.MESH,
            )
            rdma.start()
            rdma.wait()

    return pl.pallas_call(
        body,
        out_shape=jax.ShapeDtypeStruct((m, n), jnp.float32),
        in_specs=[
            pl.BlockSpec(memory_space=pltpu.VMEM),
            pl.BlockSpec(memory_space=pltpu.VMEM),
            pl.BlockSpec(memory_space=pltpu.SMEM),
            pl.BlockSpec(memory_space=pltpu.SMEM),
        ],
        out_specs=pl.BlockSpec(memory_space=pltpu.VMEM),
        scratch_shapes=[
            pltpu.VMEM((4, ch, n), jnp.float32),
            pltpu.SemaphoreType.DMA((N_DEV - 1,)),
            pltpu.SemaphoreType.DMA((N_DEV - 1,)),
            pltpu.SemaphoreType.DMA((N_DEV - 1,)),
            pltpu.SemaphoreType.DMA((N_DEV - 1,)),
            pltpu.SemaphoreType.REGULAR,
        ],
        compiler_params=pltpu.CompilerParams(
            collective_id=0,
            vmem_limit_bytes=120 * 1024 * 1024,
        ),
    )(x, w_mat, scale_x.reshape(1, 1), scale_w.reshape(1, 1))


# device time: 319769 ns/iter; 2.4336x vs baseline; 2.4336x over previous
import jax
import jax.numpy as jnp
from jax import lax
from jax.experimental import pallas as pl
from jax.experimental.pallas import tpu as pltpu

N_DEV = 16
NSLOTS = 4


def kernel(x, w_mat, scale_x, scale_w):
    m, _ = x.shape
    _, n = w_mat.shape
    ch = m // N_DEV
    hn = n // 2

    def body(x_ref, w_ref, sx_ref, sw_ref, out_ref, part_ref,
             rs_comm_cw, rs_comm_ccw, ag_comm_cw, ag_comm_ccw,
             rs_send_cw, rs_recv_cw, rs_send_ccw, rs_recv_ccw,
             ag_send_cw, ag_recv_cw, ag_send_ccw, ag_recv_ccw,
             credits):
        my = lax.axis_index("i")
        left = (my - 1 + N_DEV) % N_DEV
        right = (my + 1) % N_DEV

        barrier = pltpu.get_barrier_semaphore()
        for nbr in (left, right):
            pl.semaphore_signal(barrier, inc=1, device_id=(nbr,),
                                device_id_type=pl.DeviceIdType.MESH)
        pl.semaphore_wait(barrier, 2)

        part_ref[...] = jnp.dot(
            x_ref[...], w_ref[...], preferred_element_type=jnp.float32
        ).astype(jnp.bfloat16)

        def rows(c):
            return pl.ds(c * ch, ch)

        CW = dict(half=pl.ds(0, hn), to=right)
        CCW = dict(half=pl.ds(hn, hn), to=left)

        for s in range(N_DEV - 1):
            slot = s % NSLOTS
            if s >= NSLOTS:
                pl.semaphore_wait(credits.at[0], 1)
                pl.semaphore_wait(credits.at[1], 1)
            cw_c = (my - s + N_DEV) % N_DEV
            ccw_c = (my + s) % N_DEV
            rd_cw = pltpu.make_async_remote_copy(
                src_ref=part_ref.at[rows(cw_c), CW["half"]],
                dst_ref=rs_comm_cw.at[slot],
                send_sem=rs_send_cw.at[s], recv_sem=rs_recv_cw.at[s],
                device_id=(right,), device_id_type=pl.DeviceIdType.MESH,
            )
            rd_ccw = pltpu.make_async_remote_copy(
                src_ref=part_ref.at[rows(ccw_c), CCW["half"]],
                dst_ref=rs_comm_ccw.at[slot],
                send_sem=rs_send_ccw.at[s], recv_sem=rs_recv_ccw.at[s],
                device_id=(left,), device_id_type=pl.DeviceIdType.MESH,
            )
            rd_cw.start()
            rd_ccw.start()
            rd_cw.wait()
            rd_ccw.wait()
            acw = (my - s - 1 + N_DEV) % N_DEV
            accw = (my + s + 1) % N_DEV
            part_ref[rows(acw), CW["half"]] = (
                part_ref[rows(acw), CW["half"]] + rs_comm_cw[slot]
            )
            part_ref[rows(accw), CCW["half"]] = (
                part_ref[rows(accw), CCW["half"]] + rs_comm_ccw[slot]
            )
            if s < (N_DEV - 1) - NSLOTS:
                pl.semaphore_signal(credits.at[0], inc=1, device_id=(left,),
                                    device_id_type=pl.DeviceIdType.MESH)
                pl.semaphore_signal(credits.at[1], inc=1, device_id=(right,),
                                    device_id_type=pl.DeviceIdType.MESH)

        scale = sx_ref[0, 0] * sw_ref[0, 0]
        own_cw = (my + 1) % N_DEV
        own_ccw = (my - 1 + N_DEV) % N_DEV
        for own, d in ((own_cw, CW), (own_ccw, CCW)):
            v = jnp.maximum(
                part_ref[rows(own), d["half"]].astype(jnp.float32) * scale,
                0.0,
            )
            out_ref[rows(own), d["half"]] = v
            part_ref[rows(own), d["half"]] = v.astype(jnp.bfloat16)

        for s in range(N_DEV - 1):
            slot = s % NSLOTS
            if s >= NSLOTS:
                pl.semaphore_wait(credits.at[2], 1)
                pl.semaphore_wait(credits.at[3], 1)
            if s == 0:
                src_cw = part_ref.at[rows(own_cw), CW["half"]]
                src_ccw = part_ref.at[rows(own_ccw), CCW["half"]]
            else:
                src_cw = ag_comm_cw.at[(s - 1) % NSLOTS]
                src_ccw = ag_comm_ccw.at[(s - 1) % NSLOTS]
            rd_cw = pltpu.make_async_remote_copy(
                src_ref=src_cw,
                dst_ref=ag_comm_cw.at[slot],
                send_sem=ag_send_cw.at[s], recv_sem=ag_recv_cw.at[s],
                device_id=(right,), device_id_type=pl.DeviceIdType.MESH,
            )
            rd_ccw = pltpu.make_async_remote_copy(
                src_ref=src_ccw,
                dst_ref=ag_comm_ccw.at[slot],
                send_sem=ag_send_ccw.at[s], recv_sem=ag_recv_ccw.at[s],
                device_id=(left,), device_id_type=pl.DeviceIdType.MESH,
            )
            rd_cw.start()
            rd_ccw.start()
            rd_cw.wait()
            rd_ccw.wait()
            gcw = (my - s + N_DEV) % N_DEV
            gccw = (my + s) % N_DEV
            out_ref[rows(gcw), CW["half"]] = ag_comm_cw[slot].astype(
                jnp.float32)
            out_ref[rows(gccw), CCW["half"]] = ag_comm_ccw[slot].astype(
                jnp.float32)
            if 1 <= s <= (N_DEV - 1) - NSLOTS:
                pl.semaphore_signal(credits.at[2], inc=1, device_id=(left,),
                                    device_id_type=pl.DeviceIdType.MESH)
                pl.semaphore_signal(credits.at[3], inc=1, device_id=(right,),
                                    device_id_type=pl.DeviceIdType.MESH)

    return pl.pallas_call(
        body,
        out_shape=jax.ShapeDtypeStruct((m, n), jnp.float32),
        in_specs=[
            pl.BlockSpec(memory_space=pltpu.VMEM),
            pl.BlockSpec(memory_space=pltpu.VMEM),
            pl.BlockSpec(memory_space=pltpu.SMEM),
            pl.BlockSpec(memory_space=pltpu.SMEM),
        ],
        out_specs=pl.BlockSpec(memory_space=pltpu.VMEM),
        scratch_shapes=[
            pltpu.VMEM((m, n), jnp.bfloat16),
            pltpu.VMEM((NSLOTS, ch, hn), jnp.bfloat16),
            pltpu.VMEM((NSLOTS, ch, hn), jnp.bfloat16),
            pltpu.VMEM((NSLOTS, ch, hn), jnp.bfloat16),
            pltpu.VMEM((NSLOTS, ch, hn), jnp.bfloat16),
            pltpu.SemaphoreType.DMA((N_DEV - 1,)),
            pltpu.SemaphoreType.DMA((N_DEV - 1,)),
            pltpu.SemaphoreType.DMA((N_DEV - 1,)),
            pltpu.SemaphoreType.DMA((N_DEV - 1,)),
            pltpu.SemaphoreType.DMA((N_DEV - 1,)),
            pltpu.SemaphoreType.DMA((N_DEV - 1,)),
            pltpu.SemaphoreType.DMA((N_DEV - 1,)),
            pltpu.SemaphoreType.DMA((N_DEV - 1,)),
            pltpu.SemaphoreType.REGULAR((4,)),
        ],
        compiler_params=pltpu.CompilerParams(
            collective_id=0,
            vmem_limit_bytes=120 * 1024 * 1024,
        ),
    )(x.astype(jnp.bfloat16), w_mat.astype(jnp.bfloat16),
      scale_x.reshape(1, 1), scale_w.reshape(1, 1))


# device time: 281975 ns/iter; 2.7598x vs baseline; 1.1340x over previous
import jax
import jax.numpy as jnp
from jax import lax
from jax.experimental import pallas as pl
from jax.experimental.pallas import tpu as pltpu

N_DEV = 16
NSLOTS = 4

RING = [0, 1, 5, 4, 8, 9, 13, 12, 15, 14, 10, 11, 7, 6, 2, 3]
INV = [0] * N_DEV
for _r, _lg in enumerate(RING):
    INV[_lg] = _r


def kernel(x, w_mat, scale_x, scale_w):
    m, _ = x.shape
    _, n = w_mat.shape
    ch = m // N_DEV
    hn = n // 2

    my_log = lax.axis_index("i")
    ring_arr = jnp.asarray(RING, dtype=jnp.int32)
    inv_arr = jnp.asarray(INV, dtype=jnp.int32)
    kpos = inv_arr[my_log]
    right_log = ring_arr[(kpos + 1) % N_DEV]
    left_log = ring_arr[(kpos + N_DEV - 1) % N_DEV]
    pos = jnp.stack([kpos, left_log, right_log]).astype(jnp.int32)
    pos = pos.reshape(3, 1)

    def body(x_ref, w_ref, sx_ref, sw_ref, pos_ref, out_ref, part_ref,
             rs_comm_cw, rs_comm_ccw, ag_comm_cw, ag_comm_ccw,
             rs_send_cw, rs_recv_cw, rs_send_ccw, rs_recv_ccw,
             ag_send_cw, ag_recv_cw, ag_send_ccw, ag_recv_ccw,
             credits):
        my = pos_ref[0, 0]
        left = pos_ref[1, 0]
        right = pos_ref[2, 0]

        barrier = pltpu.get_barrier_semaphore()
        for nbr in (left, right):
            pl.semaphore_signal(barrier, inc=1, device_id=(nbr,),
                                device_id_type=pl.DeviceIdType.MESH)
        pl.semaphore_wait(barrier, 2)

        part_ref[...] = jnp.dot(
            x_ref[...], w_ref[...], preferred_element_type=jnp.float32
        ).astype(jnp.bfloat16)

        def rows(c):
            return pl.ds(c * ch, ch)

        CW = dict(half=pl.ds(0, hn), to=right)
        CCW = dict(half=pl.ds(hn, hn), to=left)

        for s in range(N_DEV - 1):
            slot = s % NSLOTS
            if s >= NSLOTS:
                pl.semaphore_wait(credits.at[0], 1)
                pl.semaphore_wait(credits.at[1], 1)
            cw_c = (my - s + N_DEV) % N_DEV
            ccw_c = (my + s) % N_DEV
            rd_cw = pltpu.make_async_remote_copy(
                src_ref=part_ref.at[rows(cw_c), CW["half"]],
                dst_ref=rs_comm_cw.at[slot],
                send_sem=rs_send_cw.at[s], recv_sem=rs_recv_cw.at[s],
                device_id=(right,), device_id_type=pl.DeviceIdType.MESH,
            )
            rd_ccw = pltpu.make_async_remote_copy(
                src_ref=part_ref.at[rows(ccw_c), CCW["half"]],
                dst_ref=rs_comm_ccw.at[slot],
                send_sem=rs_send_ccw.at[s], recv_sem=rs_recv_ccw.at[s],
                device_id=(left,), device_id_type=pl.DeviceIdType.MESH,
            )
            rd_cw.start()
            rd_ccw.start()
            rd_cw.wait()
            rd_ccw.wait()
            acw = (my - s - 1 + N_DEV) % N_DEV
            accw = (my + s + 1) % N_DEV
            part_ref[rows(acw), CW["half"]] = (
                part_ref[rows(acw), CW["half"]] + rs_comm_cw[slot]
            )
            part_ref[rows(accw), CCW["half"]] = (
                part_ref[rows(accw), CCW["half"]] + rs_comm_ccw[slot]
            )
            if s < (N_DEV - 1) - NSLOTS:
                pl.semaphore_signal(credits.at[0], inc=1, device_id=(left,),
                                    device_id_type=pl.DeviceIdType.MESH)
                pl.semaphore_signal(credits.at[1], inc=1, device_id=(right,),
                                    device_id_type=pl.DeviceIdType.MESH)

        scale = sx_ref[0, 0] * sw_ref[0, 0]
        own_cw = (my + 1) % N_DEV
        own_ccw = (my - 1 + N_DEV) % N_DEV
        for own, d in ((own_cw, CW), (own_ccw, CCW)):
            v = jnp.maximum(
                part_ref[rows(own), d["half"]].astype(jnp.float32) * scale,
                0.0,
            )
            out_ref[rows(own), d["half"]] = v
            part_ref[rows(own), d["half"]] = v.astype(jnp.bfloat16)

        for s in range(N_DEV - 1):
            slot = s % NSLOTS
            if s >= NSLOTS:
                pl.semaphore_wait(credits.at[2], 1)
                pl.semaphore_wait(credits.at[3], 1)
            if s == 0:
                src_cw = part_ref.at[rows(own_cw), CW["half"]]
                src_ccw = part_ref.at[rows(own_ccw), CCW["half"]]
            else:
                src_cw = ag_comm_cw.at[(s - 1) % NSLOTS]
                src_ccw = ag_comm_ccw.at[(s - 1) % NSLOTS]
            rd_cw = pltpu.make_async_remote_copy(
                src_ref=src_cw,
                dst_ref=ag_comm_cw.at[slot],
                send_sem=ag_send_cw.at[s], recv_sem=ag_recv_cw.at[s],
                device_id=(right,), device_id_type=pl.DeviceIdType.MESH,
            )
            rd_ccw = pltpu.make_async_remote_copy(
                src_ref=src_ccw,
                dst_ref=ag_comm_ccw.at[slot],
                send_sem=ag_send_ccw.at[s], recv_sem=ag_recv_ccw.at[s],
                device_id=(left,), device_id_type=pl.DeviceIdType.MESH,
            )
            rd_cw.start()
            rd_ccw.start()
            rd_cw.wait()
            rd_ccw.wait()
            gcw = (my - s + N_DEV) % N_DEV
            gccw = (my + s) % N_DEV
            out_ref[rows(gcw), CW["half"]] = ag_comm_cw[slot].astype(
                jnp.float32)
            out_ref[rows(gccw), CCW["half"]] = ag_comm_ccw[slot].astype(
                jnp.float32)
            if 1 <= s <= (N_DEV - 1) - NSLOTS:
                pl.semaphore_signal(credits.at[2], inc=1, device_id=(left,),
                                    device_id_type=pl.DeviceIdType.MESH)
                pl.semaphore_signal(credits.at[3], inc=1, device_id=(right,),
                                    device_id_type=pl.DeviceIdType.MESH)

    return pl.pallas_call(
        body,
        out_shape=jax.ShapeDtypeStruct((m, n), jnp.float32),
        in_specs=[
            pl.BlockSpec(memory_space=pltpu.VMEM),
            pl.BlockSpec(memory_space=pltpu.VMEM),
            pl.BlockSpec(memory_space=pltpu.SMEM),
            pl.BlockSpec(memory_space=pltpu.SMEM),
            pl.BlockSpec(memory_space=pltpu.SMEM),
        ],
        out_specs=pl.BlockSpec(memory_space=pltpu.VMEM),
        scratch_shapes=[
            pltpu.VMEM((m, n), jnp.bfloat16),
            pltpu.VMEM((NSLOTS, ch, hn), jnp.bfloat16),
            pltpu.VMEM((NSLOTS, ch, hn), jnp.bfloat16),
            pltpu.VMEM((NSLOTS, ch, hn), jnp.bfloat16),
            pltpu.VMEM((NSLOTS, ch, hn), jnp.bfloat16),
            pltpu.SemaphoreType.DMA((N_DEV - 1,)),
            pltpu.SemaphoreType.DMA((N_DEV - 1,)),
            pltpu.SemaphoreType.DMA((N_DEV - 1,)),
            pltpu.SemaphoreType.DMA((N_DEV - 1,)),
            pltpu.SemaphoreType.DMA((N_DEV - 1,)),
            pltpu.SemaphoreType.DMA((N_DEV - 1,)),
            pltpu.SemaphoreType.DMA((N_DEV - 1,)),
            pltpu.SemaphoreType.DMA((N_DEV - 1,)),
            pltpu.SemaphoreType.REGULAR((4,)),
        ],
        compiler_params=pltpu.CompilerParams(
            collective_id=0,
            vmem_limit_bytes=120 * 1024 * 1024,
        ),
    )(x.astype(jnp.bfloat16), w_mat.astype(jnp.bfloat16),
      scale_x.reshape(1, 1), scale_w.reshape(1, 1), pos)


# device time: 231462 ns/iter; 3.3621x vs baseline; 1.2182x over previous
import jax
import jax.numpy as jnp
from jax import lax
from jax.experimental import pallas as pl
from jax.experimental.pallas import tpu as pltpu

N_DEV = 16
NSLOTS = 4
SUB = 2

RING = [0, 1, 5, 4, 8, 9, 13, 12, 15, 14, 10, 11, 7, 6, 2, 3]
INV = [0] * N_DEV
for _r, _lg in enumerate(RING):
    INV[_lg] = _r


def kernel(x, w_mat, scale_x, scale_w):
    m, _ = x.shape
    _, n = w_mat.shape
    ch = m // N_DEV
    hn = n // 2
    sb = ch // SUB

    my_log = lax.axis_index("i")
    ring_arr = jnp.asarray(RING, dtype=jnp.int32)
    inv_arr = jnp.asarray(INV, dtype=jnp.int32)
    kpos = inv_arr[my_log]
    right_log = ring_arr[(kpos + 1) % N_DEV]
    left_log = ring_arr[(kpos + N_DEV - 1) % N_DEV]
    pos = jnp.stack([kpos, left_log, right_log]).astype(jnp.int32)
    pos = pos.reshape(3, 1)

    def body(x_ref, w_ref, sx_ref, sw_ref, pos_ref, out_ref, part_ref,
             rs_comm_cw, rs_comm_ccw, ag_comm_cw, ag_comm_ccw,
             rs_send_cw, rs_recv_cw, rs_send_ccw, rs_recv_ccw,
             ag_send_cw, ag_recv_cw, ag_send_ccw, ag_recv_ccw,
             credits):
        my = pos_ref[0, 0]
        left = pos_ref[1, 0]
        right = pos_ref[2, 0]

        barrier = pltpu.get_barrier_semaphore()
        for nbr in (left, right):
            pl.semaphore_signal(barrier, inc=1, device_id=(nbr,),
                                device_id_type=pl.DeviceIdType.MESH)
        pl.semaphore_wait(barrier, 2)

        part_ref[...] = jnp.dot(
            x_ref[...], w_ref[...], preferred_element_type=jnp.float32
        ).astype(jnp.bfloat16)

        def subrows(c, b):
            return pl.ds(c * ch + b * sb, sb)

        flows = []
        for b in range(SUB):
            flows.append(dict(
                b=b, half=pl.ds(0, hn), to=right, sgn=+1,
                comm=rs_comm_cw, send=rs_send_cw, recv=rs_recv_cw,
                agcomm=ag_comm_cw, agsend=ag_send_cw, agrecv=ag_recv_cw,
                rs_credit=0 * SUB + b, ag_credit=2 * SUB + b,
                credit_to=left,
            ))
            flows.append(dict(
                b=b, half=pl.ds(hn, hn), to=left, sgn=-1,
                comm=rs_comm_ccw, send=rs_send_ccw, recv=rs_recv_ccw,
                agcomm=ag_comm_ccw, agsend=ag_send_ccw, agrecv=ag_recv_ccw,
                rs_credit=1 * SUB + b, ag_credit=3 * SUB + b,
                credit_to=right,
            ))

        def rs_rdma(f, s):
            c = (my + f["sgn"] * (-s) + N_DEV) % N_DEV
            return pltpu.make_async_remote_copy(
                src_ref=part_ref.at[subrows(c, f["b"]), f["half"]],
                dst_ref=f["comm"].at[f["b"], s % NSLOTS],
                send_sem=f["send"].at[f["b"], s],
                recv_sem=f["recv"].at[f["b"], s],
                device_id=(f["to"],),
                device_id_type=pl.DeviceIdType.MESH,
            )

        for f in flows:
            rs_rdma(f, 0).start()
        for s in range(N_DEV - 1):
            for f in flows:
                rs_rdma(f, s).wait()
                ac = (my + f["sgn"] * (-s - 1) + N_DEV) % N_DEV
                part_ref[subrows(ac, f["b"]), f["half"]] = (
                    part_ref[subrows(ac, f["b"]), f["half"]]
                    + f["comm"][f["b"], s % NSLOTS]
                )
                if s < N_DEV - 2:
                    if s + 1 >= NSLOTS:
                        pl.semaphore_wait(credits.at[f["rs_credit"]], 1)
                    rs_rdma(f, s + 1).start()
                if s < (N_DEV - 1) - NSLOTS:
                    pl.semaphore_signal(
                        credits.at[f["rs_credit"]], inc=1,
                        device_id=(f["credit_to"],),
                        device_id_type=pl.DeviceIdType.MESH)

        scale = sx_ref[0, 0] * sw_ref[0, 0]
        own_cw = (my + 1) % N_DEV
        own_ccw = (my - 1 + N_DEV) % N_DEV
        for own, half in ((own_cw, pl.ds(0, hn)), (own_ccw, pl.ds(hn, hn))):
            rws = pl.ds(own * ch, ch)
            v = jnp.maximum(
                part_ref[rws, half].astype(jnp.float32) * scale, 0.0)
            out_ref[rws, half] = v
            part_ref[rws, half] = v.astype(jnp.bfloat16)

        def ag_rdma(f, s):
            own = (my + f["sgn"] + N_DEV) % N_DEV
            if s == 0:
                src = part_ref.at[subrows(own, f["b"]), f["half"]]
            else:
                src = f["agcomm"].at[f["b"], (s - 1) % NSLOTS]
            return pltpu.make_async_remote_copy(
                src_ref=src,
                dst_ref=f["agcomm"].at[f["b"], s % NSLOTS],
                send_sem=f["agsend"].at[f["b"], s],
                recv_sem=f["agrecv"].at[f["b"], s],
                device_id=(f["to"],),
                device_id_type=pl.DeviceIdType.MESH,
            )

        for f in flows:
            ag_rdma(f, 0).start()
        for s in range(N_DEV - 1):
            for f in flows:
                ag_rdma(f, s).wait()
                if s < N_DEV - 2:
                    if s + 1 >= NSLOTS:
                        pl.semaphore_wait(credits.at[f["ag_credit"]], 1)
                    ag_rdma(f, s + 1).start()
                gc = (my + f["sgn"] * (-s) + N_DEV) % N_DEV
                out_ref[subrows(gc, f["b"]), f["half"]] = (
                    f["agcomm"][f["b"], s % NSLOTS].astype(jnp.float32))
                if 1 <= s <= (N_DEV - 1) - NSLOTS:
                    pl.semaphore_signal(
                        credits.at[f["ag_credit"]], inc=1,
                        device_id=(f["credit_to"],),
                        device_id_type=pl.DeviceIdType.MESH)

    dma2 = pltpu.SemaphoreType.DMA((SUB, N_DEV - 1))
    return pl.pallas_call(
        body,
        out_shape=jax.ShapeDtypeStruct((m, n), jnp.float32),
        in_specs=[
            pl.BlockSpec(memory_space=pltpu.VMEM),
            pl.BlockSpec(memory_space=pltpu.VMEM),
            pl.BlockSpec(memory_space=pltpu.SMEM),
            pl.BlockSpec(memory_space=pltpu.SMEM),
            pl.BlockSpec(memory_space=pltpu.SMEM),
        ],
        out_specs=pl.BlockSpec(memory_space=pltpu.VMEM),
        scratch_shapes=[
            pltpu.VMEM((m, n), jnp.bfloat16),
            pltpu.VMEM((SUB, NSLOTS, sb, hn), jnp.bfloat16),
            pltpu.VMEM((SUB, NSLOTS, sb, hn), jnp.bfloat16),
            pltpu.VMEM((SUB, NSLOTS, sb, hn), jnp.bfloat16),
            pltpu.VMEM((SUB, NSLOTS, sb, hn), jnp.bfloat16),
            dma2, dma2, dma2, dma2,
            dma2, dma2, dma2, dma2,
            pltpu.SemaphoreType.REGULAR((4 * SUB,)),
        ],
        compiler_params=pltpu.CompilerParams(
            collective_id=0,
            vmem_limit_bytes=120 * 1024 * 1024,
        ),
    )(x.astype(jnp.bfloat16), w_mat.astype(jnp.bfloat16),
      scale_x.reshape(1, 1), scale_w.reshape(1, 1), pos)
